# baseline (speedup 1.0000x reference)
import jax
import jax.numpy as jnp
from jax.experimental import pallas as pl

_B, _NP, _NG, _T = 256, 128, 128, 10
_BB = 8


def _dm_kernel(gt_ref, gtt_ref, ini_ref, pred_ref, sini_ref, ppred_ref,
               mask_ref, out_ref):
    t1 = 0.0
    t2 = 0.0
    t3 = 0.0
    for b in range(_BB):
        gt = gt_ref[b]
        gx = gt[:, 0:1]
        gy = gt[:, 1:2]
        gxr = jnp.concatenate([gx[_NG - 1:_NG], gx[:_NG - 1]], axis=0)
        gyr = jnp.concatenate([gy[_NG - 1:_NG], gy[:_NG - 1]], axis=0)
        ix = ini_ref[b, 0:1, :]
        iy = ini_ref[b, 1:2, :]
        pxp = pred_ref[b, 0:1, :]
        pyp = pred_ref[b, 1:2, :]

        ex = gxr - ix
        ey = gyr - iy
        ux = gx - gxr
        uy = gy - gyr
        c0 = ex * ex + ey * ey
        c1 = 2.0 * (ex * ux + ey * uy)
        c2 = ux * ux + uy * uy

        rc2 = jnp.minimum(0.5 / c2, 1e20)
        sc = jnp.clip(c1 * (-10.0 * rc2) + 0.5, 0.0, 9.0)
        k = sc.astype(jnp.int32).astype(jnp.float32) * 0.1
        m = c0 + k * (c1 + k * c2)
        giota = jax.lax.broadcasted_iota(jnp.int32, (_NG, _NP), 0)
        mb = jax.lax.bitcast_convert_type(m, jnp.int32)
        pk = (mb & ~127) | giota
        pkmin = jnp.min(pk, axis=0, keepdims=True)
        oh = (giota == (pkmin & 127)).astype(jnp.float32)
        cx = gxr + k * ux
        cy = gyr + k * uy
        nx = jnp.sum(oh * cx, axis=0, keepdims=True)
        ny = jnp.sum(oh * cy, axis=0, keepdims=True)
        t1 = t1 + jnp.sum(jnp.abs(pxp - nx) + jnp.abs(pyp - ny))

        ixs = sini_ref[b, :, 0:1]
        iys = sini_ref[b, :, 1:2]
        pxs = ppred_ref[b, :, 0:1]
        pys = ppred_ref[b, :, 1:2]
        gxl = gtt_ref[b, 0:1, :]
        gyl = gtt_ref[b, 1:2, :]
        fx = ixs - gxl
        fy = iys - gyl
        d2 = fx * fx + fy * fy
        piota = jax.lax.broadcasted_iota(jnp.int32, (_NP, _NG), 0)
        d2b = jax.lax.bitcast_convert_type(d2, jnp.int32)
        pk2 = (d2b & ~127) | piota
        pk2min = jnp.min(pk2, axis=0, keepdims=True)
        oh2 = (piota == (pk2min & 127)).astype(jnp.float32)
        nx2 = jnp.sum(oh2 * pxs, axis=0, keepdims=True)
        ny2 = jnp.sum(oh2 * pys, axis=0, keepdims=True)
        msk = mask_ref[b]
        t2 = t2 + jnp.sum((jnp.abs(nx2 - gxl) + jnp.abs(ny2 - gyl)) * msk)
        t3 = t3 + jnp.sum(msk)

    lane = jax.lax.broadcasted_iota(jnp.int32, (1, 128), 1)
    vec = (jnp.where(lane == 0, t1, 0.0)
           + jnp.where(lane == 1, t2, 0.0)
           + jnp.where(lane == 2, t3, 0.0))

    @pl.when(pl.program_id(0) == 0)
    def _():
        out_ref[...] = jnp.zeros_like(out_ref)

    out_ref[...] += vec


@jax.jit
def kernel(ini_pred_poly, pred_polys_, gt_polys, keyPointsMask):
    ini_t = jnp.transpose(ini_pred_poly, (0, 2, 1))
    pred_t = jnp.transpose(pred_polys_, (0, 2, 1))
    gt_t = jnp.transpose(gt_polys, (0, 2, 1))
    mask3 = keyPointsMask[:, None, :]
    sums = pl.pallas_call(
        _dm_kernel,
        grid=(_B // _BB,),
        in_specs=[
            pl.BlockSpec((_BB, _NG, 2), lambda i: (i, 0, 0)),
            pl.BlockSpec((_BB, 2, _NG), lambda i: (i, 0, 0)),
            pl.BlockSpec((_BB, 2, _NP), lambda i: (i, 0, 0)),
            pl.BlockSpec((_BB, 2, _NP), lambda i: (i, 0, 0)),
            pl.BlockSpec((_BB, _NP, 2), lambda i: (i, 0, 0)),
            pl.BlockSpec((_BB, _NP, 2), lambda i: (i, 0, 0)),
            pl.BlockSpec((_BB, 1, _NG), lambda i: (i, 0, 0)),
        ],
        out_specs=pl.BlockSpec((1, 128), lambda i: (0, 0)),
        out_shape=jax.ShapeDtypeStruct((1, 128), jnp.float32),
    )(gt_polys, gt_t, ini_t, pred_t, ini_pred_poly, pred_polys_, mask3)
    t1 = sums[0, 0]
    t2 = sums[0, 1]
    t3 = sums[0, 2]
    loss1 = t1 / (_B * _NP * 2)
    loss = t2 / (2.0 * t3 + 1.0) + loss1
    return loss / 2.0
